# baseline (speedup 1.0000x reference)
import functools

import jax
import jax.numpy as jnp
from jax import lax
from jax.experimental import pallas as pl
from jax.experimental.pallas import tpu as pltpu
from jax.experimental.pallas import tpu_sc as plsc

_NC = 2
_NS = 16
_NW = _NC * _NS

_T = 160

_U = 64
_RF = 240
_RS = 80
_CH = 8


def _sc_gather_body(rf, rs, with_nodes,
                    idx_hbm, nidx_hbm, table_hbm, e_out, u_out,
                    idx_v, nidx_v, rows_v, nrows_v, gsem, wsem):
    c = lax.axis_index("c")
    s = lax.axis_index("s")
    wid = s * _NC + c
    nrpw = nidx_hbm.shape[0] // _NW
    m = nidx_hbm.shape[1]

    nun = jnp.where(c == 0, rf, rs)
    base = jnp.where(c == 0, s * rf, _NS * rf + s * rs) * _U

    if rs > 0:
        pltpu.sync_copy(idx_hbm.at[pl.ds(base, rs * _U)],
                        idx_v.at[pl.ds(0, rs * _U)])
    if rf > rs:
        @pl.when(c == 0)
        def _():
            pltpu.sync_copy(idx_hbm.at[pl.ds(base + rs * _U,
                                             (rf - rs) * _U)],
                            idx_v.at[pl.ds(rs * _U, (rf - rs) * _U)])

    def g_start(slot, i):
        pltpu.async_copy(table_hbm.at[idx_v.at[pl.ds(i * _U, _U)]],
                         rows_v.at[slot], gsem.at[slot])

    def g_wait(slot):
        pltpu.make_async_copy(table_hbm.at[idx_v.at[pl.ds(0, _U)]],
                              rows_v.at[slot], gsem.at[slot]).wait()

    def w_start(slot, i):
        pltpu.async_copy(rows_v.at[slot],
                         e_out.at[pl.ds(base + i * _U, _U)],
                         wsem.at[slot])

    def w_wait(slot):
        pltpu.make_async_copy(rows_v.at[slot], e_out.at[pl.ds(0, _U)],
                              wsem.at[slot]).wait()

    def unit(i, slot):
        g_wait(slot)
        w_start(slot, i)
        w_wait(slot)

        @pl.when(i + 2 < nun)
        def _():
            g_start(slot, i + 2)

    def outer(jo, carry):
        unit(jo * 2, 0)
        unit(jo * 2 + 1, 1)
        return carry

    @pl.when(nun > 0)
    def _():
        g_start(0, 0)
        g_start(1, 1)

    lax.fori_loop(0, (nun + 1) // 2, outer, 0, unroll=False)

    if with_nodes:
        nbase = wid * nrpw
        pltpu.sync_copy(nidx_hbm.at[pl.ds(nbase, nrpw)], nidx_v)

        def ubody(j, carry):
            pltpu.async_copy(table_hbm.at[nidx_v.at[j]], nrows_v,
                             gsem.at[0]).wait()
            pltpu.sync_copy(nrows_v, u_out.at[pl.ds((nbase + j) * m, m)])
            return carry

        lax.fori_loop(0, nrpw, ubody, 0, unroll=False)


def _tc_body(e_ref, u_ref, w1a_ref, w1b_ref, b1_ref, w2_ref, b2_ref,
             w3_ref, o_ref):
    t = u_ref.shape[0]
    k = e_ref.shape[0] // t
    d = e_ref.shape[1]
    e = e_ref[...]
    eb = e.astype(jnp.bfloat16)
    a = jnp.dot(eb, w1a_ref[...], preferred_element_type=jnp.float32)
    ub = u_ref[...].astype(jnp.bfloat16)
    c = jnp.dot(ub, w1b_ref[...],
                preferred_element_type=jnp.float32)
    h = a.reshape(t, k, d) + c[:, None, :] + b1_ref[...]
    h = jnp.maximum(h, 0.0).reshape(t * k, d).astype(jnp.bfloat16)
    h2 = jnp.dot(h, w2_ref[...], preferred_element_type=jnp.float32)
    h2 = jnp.maximum(h2 + b2_ref[...], 0.0).astype(jnp.bfloat16)
    s = jnp.dot(h2, w3_ref[...],
                preferred_element_type=jnp.float32)
    s3 = s.reshape(t, k, 1)
    m = jnp.max(s3, axis=1, keepdims=True)
    w = jnp.exp(s3 - m)
    att = w / jnp.sum(w, axis=1, keepdims=True)
    o_ref[...] = jnp.sum(att * e.reshape(t, k, d), axis=1)


def kernel(nodes, to_neighs, u2e, W1, b1, W2, b2, W3, b3):
    n, k = to_neighs.shape
    v, d = u2e.shape

    npad = ((n + 2 * _T - 1) // (2 * _T)) * (2 * _T)
    m = npad // (8 * _NW)
    assert npad * k % (128 * _NW) == 0 and npad % (8 * _NW) == 0
    assert m % 8 == 0 and m <= 128
    assert npad * k == _NS * (_RF + _RS) * _U
    assert _RF % 2 == 0 and _RS % 2 == 0 and _U % 8 == 0

    neigh_pad = jnp.zeros((npad, k), jnp.int32).at[:n].set(to_neighs)
    nodes_pad = jnp.zeros((npad,), jnp.int32).at[:n].set(nodes)
    nidx2d = nodes_pad.reshape(8 * _NW, m)

    mesh = plsc.VectorSubcoreMesh(core_axis_name="c", subcore_axis_name="s",
                                  num_cores=_NC, num_subcores=_NS)

    npc = npad // _CH
    rf, rs = _RF // _CH, _RS // _CH
    assert npc * k == _NS * (rf + rs) * _U and npc % _T == 0
    assert rf % 2 == 0 and rs % 2 == 0

    def sc_chunk(idx_chunk, with_nodes, rf, rs):
        body = functools.partial(_sc_gather_body, rf, rs, with_nodes)
        return pl.kernel(
            body,
            out_type=(jax.ShapeDtypeStruct((npc * k, d), jnp.float32),
                      jax.ShapeDtypeStruct((npad, d), jnp.float32)),
            mesh=mesh,
            scratch_types=[
                pltpu.VMEM((rf * _U,), jnp.int32),
                pltpu.VMEM((nidx2d.shape[0] // _NW, m), jnp.int32),
                pltpu.VMEM((2, _U, d), jnp.float32),
                pltpu.VMEM((m, d), jnp.float32),
                pltpu.SemaphoreType.DMA((2,)),
                pltpu.SemaphoreType.DMA((2,)),
            ],
        )(idx_chunk, nidx2d, u2e)

    grid = npc // _T
    full = lambda i: (0, 0)

    def tc_chunk(e_u, u_rep_c):
        return pl.pallas_call(
            _tc_body,
            grid=(grid,),
            in_specs=[
                pl.BlockSpec((_T * k, d), lambda i: (i, 0)),
                pl.BlockSpec((_T, d), lambda i: (i, 0)),
                pl.BlockSpec((d, d), full),
                pl.BlockSpec((d, d), full),
                pl.BlockSpec((1, d), full),
                pl.BlockSpec((d, d), full),
                pl.BlockSpec((1, d), full),
                pl.BlockSpec((d, 1), full),
            ],
            out_specs=pl.BlockSpec((_T, d), lambda i: (i, 0)),
            out_shape=jax.ShapeDtypeStruct((npc, d), jnp.float32),
        )(e_u, u_rep_c, W1[:d].astype(jnp.bfloat16),
          W1[d:].astype(jnp.bfloat16), b1.reshape(1, d),
          W2.astype(jnp.bfloat16), b2.reshape(1, d),
          W3.astype(jnp.bfloat16))

    outs = []
    u_rep = None
    for ci in range(_CH):
        idx_chunk = lax.slice_in_dim(neigh_pad, ci * npc, (ci + 1) * npc,
                                     axis=0).reshape(npc * k)
        e_u, u_rep_i = sc_chunk(idx_chunk, with_nodes=(ci == 0),
                                rf=rf, rs=rs)
        if ci == 0:
            u_rep = u_rep_i
        outs.append((e_u, lax.slice_in_dim(u_rep, ci * npc, (ci + 1) * npc,
                                           axis=0)))
    out = jnp.concatenate([tc_chunk(e, u) for e, u in outs], axis=0)
    return out[:n]


# speedup vs baseline: 1.0724x; 1.0724x over previous
import functools

import jax
import jax.numpy as jnp
from jax import lax
from jax.experimental import pallas as pl
from jax.experimental.pallas import tpu as pltpu
from jax.experimental.pallas import tpu_sc as plsc

_NC = 2
_NS = 16
_NW = _NC * _NS

_T = 160

_U = 128
_RF = 120
_RS = 40
_CH = 4


def _sc_gather_body(rf, rs, with_nodes,
                    idx_hbm, nidx_hbm, table_hbm, e_out, u_out,
                    idx_v, nidx_v, rows_v, nrows_v, gsem, wsem):
    c = lax.axis_index("c")
    s = lax.axis_index("s")
    wid = s * _NC + c
    nrpw = nidx_hbm.shape[0] // _NW
    m = nidx_hbm.shape[1]

    nun = jnp.where(c == 0, rf, rs)
    base = jnp.where(c == 0, s * rf, _NS * rf + s * rs) * _U

    if rs > 0:
        pltpu.sync_copy(idx_hbm.at[pl.ds(base, rs * _U)],
                        idx_v.at[pl.ds(0, rs * _U)])
    if rf > rs:
        @pl.when(c == 0)
        def _():
            pltpu.sync_copy(idx_hbm.at[pl.ds(base + rs * _U,
                                             (rf - rs) * _U)],
                            idx_v.at[pl.ds(rs * _U, (rf - rs) * _U)])

    def g_start(slot, i):
        pltpu.async_copy(table_hbm.at[idx_v.at[pl.ds(i * _U, _U)]],
                         rows_v.at[slot], gsem.at[slot])

    def g_wait(slot):
        pltpu.make_async_copy(table_hbm.at[idx_v.at[pl.ds(0, _U)]],
                              rows_v.at[slot], gsem.at[slot]).wait()

    def w_start(slot, i):
        pltpu.async_copy(rows_v.at[slot],
                         e_out.at[pl.ds(base + i * _U, _U)],
                         wsem.at[slot])

    def w_wait(slot):
        pltpu.make_async_copy(rows_v.at[slot], e_out.at[pl.ds(0, _U)],
                              wsem.at[slot]).wait()

    def unit(i, slot):
        g_wait(slot)
        w_start(slot, i)
        w_wait(slot)

        @pl.when(i + 2 < nun)
        def _():
            g_start(slot, i + 2)

    def outer(jo, carry):
        unit(jo * 2, 0)
        unit(jo * 2 + 1, 1)
        return carry

    @pl.when(nun > 0)
    def _():
        g_start(0, 0)
        g_start(1, 1)

    lax.fori_loop(0, (nun + 1) // 2, outer, 0, unroll=False)

    if with_nodes:
        nbase = wid * nrpw
        pltpu.sync_copy(nidx_hbm.at[pl.ds(nbase, nrpw)], nidx_v)

        def ubody(j, carry):
            pltpu.async_copy(table_hbm.at[nidx_v.at[j]], nrows_v,
                             gsem.at[0]).wait()
            pltpu.sync_copy(nrows_v, u_out.at[pl.ds((nbase + j) * m, m)])
            return carry

        lax.fori_loop(0, nrpw, ubody, 0, unroll=False)


def _tc_body(e_ref, u_ref, w1a_ref, w1b_ref, b1_ref, w2_ref, b2_ref,
             w3_ref, o_ref):
    t = u_ref.shape[0]
    k = e_ref.shape[0] // t
    d = e_ref.shape[1]
    e = e_ref[...]
    eb = e.astype(jnp.bfloat16)
    a = jnp.dot(eb, w1a_ref[...], preferred_element_type=jnp.float32)
    ub = u_ref[...].astype(jnp.bfloat16)
    c = jnp.dot(ub, w1b_ref[...],
                preferred_element_type=jnp.float32)
    h = a.reshape(t, k, d) + c[:, None, :] + b1_ref[...]
    h = jnp.maximum(h, 0.0).reshape(t * k, d).astype(jnp.bfloat16)
    h2 = jnp.dot(h, w2_ref[...], preferred_element_type=jnp.float32)
    h2 = jnp.maximum(h2 + b2_ref[...], 0.0).astype(jnp.bfloat16)
    s = jnp.dot(h2, w3_ref[...],
                preferred_element_type=jnp.float32)
    s3 = s.reshape(t, k, 1)
    m = jnp.max(s3, axis=1, keepdims=True)
    w = jnp.exp(s3 - m)
    att = w / jnp.sum(w, axis=1, keepdims=True)
    o_ref[...] = jnp.sum(att * e.reshape(t, k, d), axis=1)


def kernel(nodes, to_neighs, u2e, W1, b1, W2, b2, W3, b3):
    n, k = to_neighs.shape
    v, d = u2e.shape

    npad = ((n + 2 * _T - 1) // (2 * _T)) * (2 * _T)
    m = npad // (8 * _NW)
    assert npad * k % (128 * _NW) == 0 and npad % (8 * _NW) == 0
    assert m % 8 == 0 and m <= 128
    assert npad * k == _NS * (_RF + _RS) * _U
    assert _RF % 2 == 0 and _RS % 2 == 0 and _U % 8 == 0

    neigh_pad = jnp.zeros((npad, k), jnp.int32).at[:n].set(to_neighs)
    nodes_pad = jnp.zeros((npad,), jnp.int32).at[:n].set(nodes)
    nidx2d = nodes_pad.reshape(8 * _NW, m)

    mesh = plsc.VectorSubcoreMesh(core_axis_name="c", subcore_axis_name="s",
                                  num_cores=_NC, num_subcores=_NS)

    npc = npad // _CH
    rf, rs = _RF // _CH, _RS // _CH
    assert npc * k == _NS * (rf + rs) * _U and npc % _T == 0
    assert rf % 2 == 0 and rs % 2 == 0

    def sc_chunk(idx_chunk, with_nodes, rf, rs):
        body = functools.partial(_sc_gather_body, rf, rs, with_nodes)
        return pl.kernel(
            body,
            out_type=(jax.ShapeDtypeStruct((npc * k, d), jnp.float32),
                      jax.ShapeDtypeStruct((npad, d), jnp.float32)),
            mesh=mesh,
            scratch_types=[
                pltpu.VMEM((rf * _U,), jnp.int32),
                pltpu.VMEM((nidx2d.shape[0] // _NW, m), jnp.int32),
                pltpu.VMEM((2, _U, d), jnp.float32),
                pltpu.VMEM((m, d), jnp.float32),
                pltpu.SemaphoreType.DMA((2,)),
                pltpu.SemaphoreType.DMA((2,)),
            ],
        )(idx_chunk, nidx2d, u2e)

    grid = npc // _T
    full = lambda i: (0, 0)

    def tc_chunk(e_u, u_rep_c):
        return pl.pallas_call(
            _tc_body,
            grid=(grid,),
            in_specs=[
                pl.BlockSpec((_T * k, d), lambda i: (i, 0)),
                pl.BlockSpec((_T, d), lambda i: (i, 0)),
                pl.BlockSpec((d, d), full),
                pl.BlockSpec((d, d), full),
                pl.BlockSpec((1, d), full),
                pl.BlockSpec((d, d), full),
                pl.BlockSpec((1, d), full),
                pl.BlockSpec((d, 1), full),
            ],
            out_specs=pl.BlockSpec((_T, d), lambda i: (i, 0)),
            out_shape=jax.ShapeDtypeStruct((npc, d), jnp.float32),
        )(e_u, u_rep_c, W1[:d].astype(jnp.bfloat16),
          W1[d:].astype(jnp.bfloat16), b1.reshape(1, d),
          W2.astype(jnp.bfloat16), b2.reshape(1, d),
          W3.astype(jnp.bfloat16))

    outs = []
    u_rep = None
    for ci in range(_CH):
        idx_chunk = lax.slice_in_dim(neigh_pad, ci * npc, (ci + 1) * npc,
                                     axis=0).reshape(npc * k)
        e_u, u_rep_i = sc_chunk(idx_chunk, with_nodes=(ci == 0),
                                rf=rf, rs=rs)
        if ci == 0:
            u_rep = u_rep_i
        outs.append((e_u, lax.slice_in_dim(u_rep, ci * npc, (ci + 1) * npc,
                                           axis=0)))
    out = jnp.concatenate([tc_chunk(e, u) for e, u in outs], axis=0)
    return out[:n]
